# baseline (speedup 1.0000x reference)
import jax
import jax.numpy as jnp
from jax import lax
from jax.experimental import pallas as pl
from jax.experimental.pallas import tpu as pltpu
from jax.experimental.pallas import tpu_sc as plsc

NUM_CLASSES = 100000
DIM = 128
BATCH = 16384

NC = 2
NS = 16
NW = NC * NS
B_PER_W = BATCH // NW
CHUNK = 128
N_CHUNKS = B_PER_W // CHUNK
NSETS = 3
BB = 1024
NSLOTS = 3


def _sc_gather_body(labels_hbm, t1_hbm, t2_hbm, stk_hbm,
                    idx_v, buf1, buf2, gsem, wsem):
    wid = lax.axis_index("s") * NC + lax.axis_index("c")
    pltpu.sync_copy(labels_hbm.at[pl.ds(wid * B_PER_W, B_PER_W)], idx_v)

    gd = [None] * N_CHUNKS
    wd = [None] * N_CHUNKS

    def issue_gather(c):
        s = c % NSETS
        idx_c = idx_v.at[pl.ds(c * CHUNK, CHUNK)]
        gd[c] = (pltpu.async_copy(t1_hbm.at[idx_c], buf1.at[s], gsem.at[s]),
                 pltpu.async_copy(t2_hbm.at[idx_c], buf2.at[s], gsem.at[s]))

    for c in range(min(NSETS, N_CHUNKS)):
        issue_gather(c)

    for c in range(N_CHUNKS):
        s = c % NSETS
        gd[c][0].wait()
        gd[c][1].wait()
        rows = pl.ds((wid * N_CHUNKS + c) * CHUNK, CHUNK)
        wd[c] = (
            pltpu.async_copy(buf1.at[s], stk_hbm.at[rows, 0], wsem.at[s]),
            pltpu.async_copy(buf2.at[s], stk_hbm.at[rows, 1], wsem.at[s]),
        )
        nxt = c + NSETS
        if nxt < N_CHUNKS:
            for d in wd[c]:
                d.wait()
            wd[c] = None
            issue_gather(nxt)

    for c in range(N_CHUNKS):
        if wd[c] is not None:
            for d in wd[c]:
                d.wait()


def _sc_gather(labels1d, table1, table2):
    mesh = plsc.VectorSubcoreMesh(
        core_axis_name="c", subcore_axis_name="s",
        num_cores=NC, num_subcores=NS)
    k = pl.kernel(
        _sc_gather_body,
        out_type=jax.ShapeDtypeStruct((BATCH, 2, DIM), jnp.float32),
        mesh=mesh,
        scratch_types=[
            pltpu.VMEM((B_PER_W,), jnp.int32),
            pltpu.VMEM((NSETS, CHUNK, DIM), jnp.float32),
            pltpu.VMEM((NSETS, CHUNK, DIM), jnp.float32),
            pltpu.SemaphoreType.DMA((NSETS,)),
            pltpu.SemaphoreType.DMA((NSETS,)),
        ],
    )
    return k(labels1d, table1, table2)


def _mlp_body(stk_ref, w1_ref, b1_ref, w2_ref, b2_ref, out_ref,
              e1b, e2b, sem1, sem2):
    i = pl.program_id(0)
    n = pl.num_programs(0)

    def copies(step, slot):
        rows = pl.ds(step * BB, BB)
        return (pltpu.make_async_copy(stk_ref.at[rows, 0], e1b.at[slot],
                                      sem1.at[slot]),
                pltpu.make_async_copy(stk_ref.at[rows, 1], e2b.at[slot],
                                      sem2.at[slot]))

    slot = lax.rem(i, NSLOTS)

    @pl.when(i == 0)
    def _():
        for p in range(NSLOTS - 1):
            for c in copies(i + p, lax.rem(i + p, NSLOTS)):
                c.start()

    @pl.when(i + NSLOTS - 1 < n)
    def _():
        for c in copies(i + NSLOTS - 1, lax.rem(i + NSLOTS - 1, NSLOTS)):
            c.start()

    for c in copies(i, slot):
        c.wait()

    e1 = e1b[slot]
    e2 = e2b[slot]
    w1a = w1_ref[:DIM, :]
    w1b = w1_ref[DIM:, :]
    h = (jnp.dot(e1, w1a, preferred_element_type=jnp.float32)
         + jnp.dot(e2, w1b, preferred_element_type=jnp.float32)
         + b1_ref[0, :][None, :])
    h = h * jax.nn.sigmoid(h)
    g = jnp.dot(h, w2_ref[...], preferred_element_type=jnp.float32)
    out_ref[...] = g + b2_ref[0, :][None, :]


def _mlp(stk, W1, b1, W2, b2):
    return pl.pallas_call(
        _mlp_body,
        grid=(BATCH // BB,),
        in_specs=[
            pl.BlockSpec(memory_space=pltpu.MemorySpace.HBM),
            pl.BlockSpec((2 * DIM, DIM), lambda i: (0, 0)),
            pl.BlockSpec((1, DIM), lambda i: (0, 0)),
            pl.BlockSpec((DIM, DIM), lambda i: (0, 0)),
            pl.BlockSpec((1, DIM), lambda i: (0, 0)),
        ],
        out_specs=pl.BlockSpec((BB, DIM), lambda i: (i, 0)),
        out_shape=jax.ShapeDtypeStruct((BATCH, DIM), jnp.float32),
        scratch_shapes=[
            pltpu.VMEM((NSLOTS, BB, DIM), jnp.float32),
            pltpu.VMEM((NSLOTS, BB, DIM), jnp.float32),
            pltpu.SemaphoreType.DMA((NSLOTS,)),
            pltpu.SemaphoreType.DMA((NSLOTS,)),
        ],
    )(stk, W1, b1, W2, b2)


def kernel(labels, train, table1, table2, W1, b1, W2, b2):
    labels1d = labels.astype(jnp.int32)
    embeddings = _sc_gather(labels1d, table1, table2)
    global_embeddings = _mlp(embeddings, W1, b1.reshape(1, DIM),
                             W2, b2.reshape(1, DIM))
    return (embeddings, global_embeddings)


# speedup vs baseline: 1.0073x; 1.0073x over previous
import jax
import jax.numpy as jnp
from jax import lax
from jax.experimental import pallas as pl
from jax.experimental.pallas import tpu as pltpu
from jax.experimental.pallas import tpu_sc as plsc

NUM_CLASSES = 100000
DIM = 128
BATCH = 16384

NC = 2
NS = 16
NW = NC * NS
B_PER_W = BATCH // NW
CHUNK = 64
N_CHUNKS = B_PER_W // CHUNK
NSETS = 6
BB = 2048
NSLOTS = 2


def _sc_gather_body(labels_hbm, t1_hbm, t2_hbm, stk_hbm,
                    idx_v, buf1, buf2, gsem, wsem):
    wid = lax.axis_index("s") * NC + lax.axis_index("c")
    pltpu.sync_copy(labels_hbm.at[pl.ds(wid * B_PER_W, B_PER_W)], idx_v)

    gd = [None] * N_CHUNKS
    wd = [None] * N_CHUNKS

    def issue_gather(c):
        s = c % NSETS
        idx_c = idx_v.at[pl.ds(c * CHUNK, CHUNK)]
        gd[c] = (pltpu.async_copy(t1_hbm.at[idx_c], buf1.at[s], gsem.at[s]),
                 pltpu.async_copy(t2_hbm.at[idx_c], buf2.at[s], gsem.at[s]))

    for c in range(min(NSETS, N_CHUNKS)):
        issue_gather(c)

    for c in range(N_CHUNKS):
        s = c % NSETS
        gd[c][0].wait()
        gd[c][1].wait()
        rows = pl.ds((wid * N_CHUNKS + c) * CHUNK, CHUNK)
        wd[c] = (
            pltpu.async_copy(buf1.at[s], stk_hbm.at[rows, 0], wsem.at[s]),
            pltpu.async_copy(buf2.at[s], stk_hbm.at[rows, 1], wsem.at[s]),
        )
        nxt = c + NSETS
        if nxt < N_CHUNKS:
            for d in wd[c]:
                d.wait()
            wd[c] = None
            issue_gather(nxt)

    for c in range(N_CHUNKS):
        if wd[c] is not None:
            for d in wd[c]:
                d.wait()


def _sc_gather(labels1d, table1, table2):
    mesh = plsc.VectorSubcoreMesh(
        core_axis_name="c", subcore_axis_name="s",
        num_cores=NC, num_subcores=NS)
    k = pl.kernel(
        _sc_gather_body,
        out_type=jax.ShapeDtypeStruct((BATCH, 2, DIM), jnp.float32),
        mesh=mesh,
        scratch_types=[
            pltpu.VMEM((B_PER_W,), jnp.int32),
            pltpu.VMEM((NSETS, CHUNK, DIM), jnp.float32),
            pltpu.VMEM((NSETS, CHUNK, DIM), jnp.float32),
            pltpu.SemaphoreType.DMA((NSETS,)),
            pltpu.SemaphoreType.DMA((NSETS,)),
        ],
    )
    return k(labels1d, table1, table2)


def _mlp_body(stk_ref, w1_ref, b1_ref, w2_ref, b2_ref, out_ref,
              e1b, e2b, sem1, sem2):
    i = pl.program_id(0)
    n = pl.num_programs(0)

    def copies(step, slot):
        rows = pl.ds(step * BB, BB)
        return (pltpu.make_async_copy(stk_ref.at[rows, 0], e1b.at[slot],
                                      sem1.at[slot]),
                pltpu.make_async_copy(stk_ref.at[rows, 1], e2b.at[slot],
                                      sem2.at[slot]))

    slot = lax.rem(i, NSLOTS)

    @pl.when(i == 0)
    def _():
        for p in range(NSLOTS - 1):
            for c in copies(i + p, lax.rem(i + p, NSLOTS)):
                c.start()

    @pl.when(i + NSLOTS - 1 < n)
    def _():
        for c in copies(i + NSLOTS - 1, lax.rem(i + NSLOTS - 1, NSLOTS)):
            c.start()

    for c in copies(i, slot):
        c.wait()

    e1 = e1b[slot]
    e2 = e2b[slot]
    w1a = w1_ref[:DIM, :]
    w1b = w1_ref[DIM:, :]
    h = (jnp.dot(e1, w1a, preferred_element_type=jnp.float32)
         + jnp.dot(e2, w1b, preferred_element_type=jnp.float32)
         + b1_ref[0, :][None, :])
    h = h * jax.nn.sigmoid(h)
    g = jnp.dot(h, w2_ref[...], preferred_element_type=jnp.float32)
    out_ref[...] = g + b2_ref[0, :][None, :]


def _mlp(stk, W1, b1, W2, b2):
    return pl.pallas_call(
        _mlp_body,
        grid=(BATCH // BB,),
        in_specs=[
            pl.BlockSpec(memory_space=pltpu.MemorySpace.HBM),
            pl.BlockSpec((2 * DIM, DIM), lambda i: (0, 0)),
            pl.BlockSpec((1, DIM), lambda i: (0, 0)),
            pl.BlockSpec((DIM, DIM), lambda i: (0, 0)),
            pl.BlockSpec((1, DIM), lambda i: (0, 0)),
        ],
        out_specs=pl.BlockSpec((BB, DIM), lambda i: (i, 0)),
        out_shape=jax.ShapeDtypeStruct((BATCH, DIM), jnp.float32),
        scratch_shapes=[
            pltpu.VMEM((NSLOTS, BB, DIM), jnp.float32),
            pltpu.VMEM((NSLOTS, BB, DIM), jnp.float32),
            pltpu.SemaphoreType.DMA((NSLOTS,)),
            pltpu.SemaphoreType.DMA((NSLOTS,)),
        ],
    )(stk, W1, b1, W2, b2)


def kernel(labels, train, table1, table2, W1, b1, W2, b2):
    labels1d = labels.astype(jnp.int32)
    embeddings = _sc_gather(labels1d, table1, table2)
    global_embeddings = _mlp(embeddings, W1, b1.reshape(1, DIM),
                             W2, b2.reshape(1, DIM))
    return (embeddings, global_embeddings)


# speedup vs baseline: 1.0173x; 1.0100x over previous
import jax
import jax.numpy as jnp
from jax import lax
from jax.experimental import pallas as pl
from jax.experimental.pallas import tpu as pltpu
from jax.experimental.pallas import tpu_sc as plsc

NUM_CLASSES = 100000
DIM = 128
BATCH = 16384

NC = 2
NS = 16
NW = NC * NS
B_PER_W = BATCH // NW
CHUNK = 128
N_CHUNKS = B_PER_W // CHUNK
NSETS = 3
BB = 2048
NSLOTS = 2


def _sc_gather_body(labels_hbm, t1_hbm, t2_hbm, stk_hbm,
                    idx_v, buf1, buf2, gsem, wsem):
    wid = lax.axis_index("s") * NC + lax.axis_index("c")
    pltpu.sync_copy(labels_hbm.at[pl.ds(wid * B_PER_W, B_PER_W)], idx_v)

    gd = [None] * N_CHUNKS
    wd = [None] * N_CHUNKS

    def issue_gather(c):
        s = c % NSETS
        idx_c = idx_v.at[pl.ds(c * CHUNK, CHUNK)]
        gd[c] = (pltpu.async_copy(t1_hbm.at[idx_c], buf1.at[s], gsem.at[s]),
                 pltpu.async_copy(t2_hbm.at[idx_c], buf2.at[s], gsem.at[s]))

    for c in range(min(NSETS, N_CHUNKS)):
        issue_gather(c)

    for c in range(N_CHUNKS):
        s = c % NSETS
        gd[c][0].wait()
        gd[c][1].wait()
        rows = pl.ds((wid * N_CHUNKS + c) * CHUNK, CHUNK)
        wd[c] = (
            pltpu.async_copy(buf1.at[s], stk_hbm.at[rows, 0], wsem.at[s]),
            pltpu.async_copy(buf2.at[s], stk_hbm.at[rows, 1], wsem.at[s]),
        )
        nxt = c + NSETS
        if nxt < N_CHUNKS:
            for d in wd[c]:
                d.wait()
            wd[c] = None
            issue_gather(nxt)

    for c in range(N_CHUNKS):
        if wd[c] is not None:
            for d in wd[c]:
                d.wait()


def _sc_gather(labels1d, table1, table2):
    mesh = plsc.VectorSubcoreMesh(
        core_axis_name="c", subcore_axis_name="s",
        num_cores=NC, num_subcores=NS)
    k = pl.kernel(
        _sc_gather_body,
        out_type=jax.ShapeDtypeStruct((BATCH, 2, DIM), jnp.float32),
        mesh=mesh,
        scratch_types=[
            pltpu.VMEM((B_PER_W,), jnp.int32),
            pltpu.VMEM((NSETS, CHUNK, DIM), jnp.float32),
            pltpu.VMEM((NSETS, CHUNK, DIM), jnp.float32),
            pltpu.SemaphoreType.DMA((NSETS,)),
            pltpu.SemaphoreType.DMA((NSETS,)),
        ],
    )
    return k(labels1d, table1, table2)


def _mlp_body(stk_ref, w1_ref, b1_ref, w2_ref, b2_ref, out_ref,
              e1b, e2b, sem1, sem2):
    i = pl.program_id(0)
    n = pl.num_programs(0)

    def copies(step, slot):
        rows = pl.ds(step * BB, BB)
        return (pltpu.make_async_copy(stk_ref.at[rows, 0], e1b.at[slot],
                                      sem1.at[slot]),
                pltpu.make_async_copy(stk_ref.at[rows, 1], e2b.at[slot],
                                      sem2.at[slot]))

    slot = lax.rem(i, NSLOTS)

    @pl.when(i == 0)
    def _():
        for p in range(NSLOTS - 1):
            for c in copies(i + p, lax.rem(i + p, NSLOTS)):
                c.start()

    @pl.when(i + NSLOTS - 1 < n)
    def _():
        for c in copies(i + NSLOTS - 1, lax.rem(i + NSLOTS - 1, NSLOTS)):
            c.start()

    for c in copies(i, slot):
        c.wait()

    e1 = e1b[slot]
    e2 = e2b[slot]
    w1a = w1_ref[:DIM, :]
    w1b = w1_ref[DIM:, :]
    h = (jnp.dot(e1, w1a, preferred_element_type=jnp.float32)
         + jnp.dot(e2, w1b, preferred_element_type=jnp.float32)
         + b1_ref[0, :][None, :])
    h = h * jax.nn.sigmoid(h)
    g = jnp.dot(h, w2_ref[...], preferred_element_type=jnp.float32)
    out_ref[...] = g + b2_ref[0, :][None, :]


def _mlp(stk, W1, b1, W2, b2):
    return pl.pallas_call(
        _mlp_body,
        grid=(BATCH // BB,),
        in_specs=[
            pl.BlockSpec(memory_space=pltpu.MemorySpace.HBM),
            pl.BlockSpec((2 * DIM, DIM), lambda i: (0, 0)),
            pl.BlockSpec((1, DIM), lambda i: (0, 0)),
            pl.BlockSpec((DIM, DIM), lambda i: (0, 0)),
            pl.BlockSpec((1, DIM), lambda i: (0, 0)),
        ],
        out_specs=pl.BlockSpec((BB, DIM), lambda i: (i, 0)),
        out_shape=jax.ShapeDtypeStruct((BATCH, DIM), jnp.float32),
        scratch_shapes=[
            pltpu.VMEM((NSLOTS, BB, DIM), jnp.float32),
            pltpu.VMEM((NSLOTS, BB, DIM), jnp.float32),
            pltpu.SemaphoreType.DMA((NSLOTS,)),
            pltpu.SemaphoreType.DMA((NSLOTS,)),
        ],
    )(stk, W1, b1, W2, b2)


def kernel(labels, train, table1, table2, W1, b1, W2, b2):
    labels1d = labels.astype(jnp.int32)
    embeddings = _sc_gather(labels1d, table1, table2)
    global_embeddings = _mlp(embeddings, W1, b1.reshape(1, DIM),
                             W2, b2.reshape(1, DIM))
    return (embeddings, global_embeddings)
